# baseline (speedup 1.0000x reference)
import functools

import jax
import jax.numpy as jnp
from jax import lax
from jax.experimental import pallas as pl
from jax.experimental.pallas import tpu as pltpu
from jax.experimental.pallas import tpu_sc as plsc

N_BINS = 15
C = 8
B = 4
N = 512 * 512
NW = 32
CHUNKS_PER_B = NW // B
PIX_W = N // CHUNKS_PER_B
SUB = 4096
NSUB = PIX_W // SUB
VEC = 16
VPS = SUB // VEC
ROWS = C * N_BINS
HWORDS = ROWS * VEC
TSHIFT = 12

_mesh = plsc.VectorSubcoreMesh(core_axis_name="c", subcore_axis_name="s")


@functools.partial(
    pl.kernel,
    out_type=jax.ShapeDtypeStruct((NW, 3 * ROWS), jnp.float32),
    mesh=_mesh,
    compiler_params=pltpu.CompilerParams(needs_layout_passes=False),
    scratch_types=[
        pltpu.VMEM((2, C, SUB), jnp.float32),
        pltpu.VMEM((PIX_W,), jnp.int32),
        pltpu.VMEM((VEC,), jnp.float32),
        pltpu.VMEM((VEC,), jnp.float32),
        pltpu.VMEM((HWORDS,), jnp.float32),
        pltpu.VMEM((HWORDS,), jnp.float32),
        pltpu.VMEM((3 * ROWS,), jnp.float32),
        pltpu.SemaphoreType.DMA,
        pltpu.SemaphoreType.DMA,
        pltpu.SemaphoreType.DMA,
    ],
)
def _sc_hist(logits_hbm, labels_hbm, bounds_hbm, parts_hbm,
             lbuf, labv, bndlo, bndhi, hist_p, hist_ct, outv,
             sem0, sem1, sem2):
    wid = lax.axis_index("s") * 2 + lax.axis_index("c")
    b = wid // CHUNKS_PER_B
    jj = wid % CHUNKS_PER_B
    base_pix = jj * PIX_W

    lab_cp = pltpu.async_copy(labels_hbm.at[b, pl.ds(base_pix, PIX_W)],
                              labv, sem2)
    pltpu.sync_copy(bounds_hbm.at[0], bndlo)
    pltpu.sync_copy(bounds_hbm.at[1], bndhi)

    sems = (sem0, sem1)

    def start(it):
        return pltpu.async_copy(
            logits_hbm.at[b, :, pl.ds(base_pix + it * SUB, SUB)],
            lbuf.at[it % 2], sems[it % 2])

    zero16 = jnp.zeros((VEC,), jnp.float32)

    def _zi(i, carry):
        hist_p[pl.ds(i * VEC, VEC)] = zero16
        hist_ct[pl.ds(i * VEC, VEC)] = zero16
        return carry

    lax.fori_loop(0, ROWS, _zi, 0)

    lane = lax.iota(jnp.int32, 16)
    eps32 = 1.1920929e-07
    inv_step = jnp.float32(15.0 / (1.0 + eps32))
    one = jnp.float32(1.0)
    tval = jnp.float32(1.0 + (1 << TSHIFT))
    lanec = [lane + c * N_BINS * VEC for c in range(C)]

    def compute(it, parity):
        sub_base = it * SUB

        @plsc.parallel_loop(0, VPS, unroll=12)
        def body(i):
            off = i * VEC
            lab = labv[pl.ds(sub_base + off, VEC)]
            xs = [lbuf[parity, c, pl.ds(off, VEC)] for c in range(C)]
            m01 = jnp.maximum(xs[0], xs[1])
            m23 = jnp.maximum(xs[2], xs[3])
            m45 = jnp.maximum(xs[4], xs[5])
            m67 = jnp.maximum(xs[6], xs[7])
            m = jnp.maximum(jnp.maximum(m01, m23), jnp.maximum(m45, m67))
            es = [jnp.exp(xs[c] - m) for c in range(C)]
            s01 = es[0] + es[1]
            s23 = es[2] + es[3]
            s45 = es[4] + es[5]
            s67 = es[6] + es[7]
            rsum = one / ((s01 + s23) + (s45 + s67))
            rs2 = rsum * inv_step
            for c in range(C):
                p = es[c] * rsum
                m0 = (es[c] * rs2).astype(jnp.int32)
                blo = plsc.load_gather(bndlo, [m0])
                bhi = plsc.load_gather(bndhi, [m0])
                corr = jnp.where(p <= blo, -1, jnp.where(p > bhi, 1, 0))
                idx = (m0 + corr) * VEC + lanec[c]
                plsc.addupdate_scatter(hist_p, [idx], p)
                tv = jnp.where(lab == c, tval, one)
                plsc.addupdate_scatter(hist_ct, [idx], tv)

    handles = [start(0), None]
    lab_cp.wait()
    for it in range(NSUB):
        if it + 1 < NSUB:
            handles[(it + 1) % 2] = start(it + 1)
        handles[it % 2].wait()
        compute(it, it % 2)

    mask_lo = jnp.int32((1 << TSHIFT) - 1)

    lane0 = lane == 0

    def _red(r, carry):
        vp = hist_p[pl.ds(r * VEC, VEC)]
        vi = hist_ct[pl.ds(r * VEC, VEC)].astype(jnp.int32)
        sp = jnp.broadcast_to(jnp.sum(vp), (VEC,))
        sc = jnp.broadcast_to(jnp.sum(vi & mask_lo), (VEC,)).astype(jnp.float32)
        st = jnp.broadcast_to(jnp.sum(vi >> TSHIFT), (VEC,)).astype(jnp.float32)
        r16 = jnp.broadcast_to(r, (VEC,))
        plsc.store_scatter(outv, [r16], sp, mask=lane0)
        plsc.store_scatter(outv, [r16 + ROWS], sc, mask=lane0)
        plsc.store_scatter(outv, [r16 + 2 * ROWS], st, mask=lane0)
        return carry

    lax.fori_loop(0, ROWS, _red, 0)
    pltpu.sync_copy(outv, parts_hbm.at[wid])


def _epi_body(parts_ref, w4_ref, k_ref, out_ref):
    e = parts_ref[...]
    a = jnp.dot(w4_ref[...], e, precision=lax.Precision.HIGHEST)
    sp = a[:, :ROWS]
    cnt = a[:, ROWS:2 * ROWS]
    st = a[:, 2 * ROWS:]
    ne = cnt > 0
    safe = jnp.where(ne, cnt, 1.0)
    mp = sp / safe
    mt = st / safe
    diff = jnp.where(ne, jnp.abs(mp - mt), 0.0)
    kk = k_ref[...]
    nv = jnp.maximum(
        jnp.dot(ne.astype(jnp.float32), kk, precision=lax.Precision.HIGHEST),
        1.0)
    dsum = jnp.dot(diff, kk, precision=lax.Precision.HIGHEST)
    stot = jnp.dot(st, kk, precision=lax.Precision.HIGHEST)
    ace = (dsum / nv) * (stot > 0).astype(jnp.float32)
    out_ref[...] = (jnp.sum(ace) / jnp.float32(B * C)).reshape(1, 1)


_epilogue = pl.pallas_call(
    _epi_body,
    out_shape=jax.ShapeDtypeStruct((1, 1), jnp.float32),
)


def kernel(logits, labels):
    lg = logits.reshape(B, C, N)
    lb = labels.reshape(B, N)
    eps = jnp.finfo(jnp.float32).eps
    bounds = jnp.linspace(0.0, 1.0 + eps, N_BINS + 1).astype(jnp.float32)
    step = jnp.float32((1.0 + float(eps)) / 15.0)
    b16 = jnp.float32(16.0) * step
    row_lo = jnp.concatenate([jnp.full((1,), -1.0, jnp.float32), bounds[1:]])
    row_hi = jnp.concatenate([bounds[1:], b16[None]])
    btab = jnp.stack([row_lo, row_hi])
    parts = _sc_hist(lg, lb, btab)
    w4 = (jnp.arange(NW, dtype=jnp.int32)[None, :] // CHUNKS_PER_B
          == jnp.arange(B, dtype=jnp.int32)[:, None]).astype(jnp.float32)
    kk = (jnp.arange(ROWS, dtype=jnp.int32)[:, None] // N_BINS
          == jnp.arange(C, dtype=jnp.int32)[None, :]).astype(jnp.float32)
    return _epilogue(parts, w4, kk).reshape(())


# speedup vs baseline: 1.0486x; 1.0486x over previous
import functools

import jax
import jax.numpy as jnp
from jax import lax
from jax.experimental import pallas as pl
from jax.experimental.pallas import tpu as pltpu
from jax.experimental.pallas import tpu_sc as plsc

N_BINS = 15
C = 8
B = 4
N = 512 * 512
NW = 32
CHUNKS_PER_B = NW // B
PIX_W = N // CHUNKS_PER_B
SUB = 4096
NSUB = PIX_W // SUB
VEC = 16
VPS = SUB // VEC
ROWS = C * N_BINS
HWORDS = ROWS * VEC
TSHIFT = 12

_mesh = plsc.VectorSubcoreMesh(core_axis_name="c", subcore_axis_name="s")


@functools.partial(
    pl.kernel,
    out_type=jax.ShapeDtypeStruct((NW, 3 * ROWS), jnp.float32),
    mesh=_mesh,
    compiler_params=pltpu.CompilerParams(needs_layout_passes=False),
    scratch_types=[
        pltpu.VMEM((2, C, SUB), jnp.float32),
        pltpu.VMEM((PIX_W,), jnp.int32),
        pltpu.VMEM((VEC,), jnp.float32),
        pltpu.VMEM((VEC,), jnp.float32),
        pltpu.VMEM((HWORDS,), jnp.float32),
        pltpu.VMEM((HWORDS,), jnp.float32),
        pltpu.VMEM((3 * ROWS,), jnp.float32),
        pltpu.SemaphoreType.DMA,
        pltpu.SemaphoreType.DMA,
        pltpu.SemaphoreType.DMA,
    ],
)
def _sc_hist(logits_hbm, labels_hbm, bounds_hbm, parts_hbm,
             lbuf, labv, bndlo, bndhi, hist_p, hist_ct, outv,
             sem0, sem1, sem2):
    wid = lax.axis_index("s") * 2 + lax.axis_index("c")
    b = wid // CHUNKS_PER_B
    jj = wid % CHUNKS_PER_B
    base_pix = jj * PIX_W

    lab_cp = pltpu.async_copy(labels_hbm.at[b, pl.ds(base_pix, PIX_W)],
                              labv, sem2)
    pltpu.sync_copy(bounds_hbm.at[0], bndlo)
    pltpu.sync_copy(bounds_hbm.at[1], bndhi)

    sems = (sem0, sem1)

    def start(it):
        return pltpu.async_copy(
            logits_hbm.at[b, :, pl.ds(base_pix + it * SUB, SUB)],
            lbuf.at[it % 2], sems[it % 2])

    zero16 = jnp.zeros((VEC,), jnp.float32)

    def _zi(i, carry):
        hist_p[pl.ds(i * VEC, VEC)] = zero16
        hist_ct[pl.ds(i * VEC, VEC)] = zero16
        return carry

    lax.fori_loop(0, ROWS, _zi, 0)

    lane = lax.iota(jnp.int32, 16)
    eps32 = 1.1920929e-07
    inv_step = jnp.float32(15.0 / (1.0 + eps32))
    one = jnp.float32(1.0)
    tval = jnp.float32(1.0 + (1 << TSHIFT))
    lanec = [lane + c * N_BINS * VEC for c in range(C)]

    def compute(it, parity):
        sub_base = it * SUB

        @plsc.parallel_loop(0, VPS, unroll=8)
        def body(i):
            off = i * VEC
            lab = labv[pl.ds(sub_base + off, VEC)]
            xs = [lbuf[parity, c, pl.ds(off, VEC)] for c in range(C)]
            m01 = jnp.maximum(xs[0], xs[1])
            m23 = jnp.maximum(xs[2], xs[3])
            m45 = jnp.maximum(xs[4], xs[5])
            m67 = jnp.maximum(xs[6], xs[7])
            m = jnp.maximum(jnp.maximum(m01, m23), jnp.maximum(m45, m67))
            es = [jnp.exp(xs[c] - m) for c in range(C)]
            s01 = es[0] + es[1]
            s23 = es[2] + es[3]
            s45 = es[4] + es[5]
            s67 = es[6] + es[7]
            rsum = one / ((s01 + s23) + (s45 + s67))
            rs2 = rsum * inv_step
            for c in range(C):
                p = es[c] * rsum
                m0 = (es[c] * rs2).astype(jnp.int32)
                blo = plsc.load_gather(bndlo, [m0])
                bhi = plsc.load_gather(bndhi, [m0])
                corr = jnp.where(p <= blo, -1, jnp.where(p > bhi, 1, 0))
                idx = (m0 + corr) * VEC + lanec[c]
                plsc.addupdate_scatter(hist_p, [idx], p)
                tv = jnp.where(lab == c, tval, one)
                plsc.addupdate_scatter(hist_ct, [idx], tv)

    handles = [start(0), None]
    lab_cp.wait()
    for it in range(NSUB):
        if it + 1 < NSUB:
            handles[(it + 1) % 2] = start(it + 1)
        handles[it % 2].wait()
        compute(it, it % 2)

    mask_lo = jnp.int32((1 << TSHIFT) - 1)

    lane0 = lane == 0

    def _red(r, carry):
        vp = hist_p[pl.ds(r * VEC, VEC)]
        vi = hist_ct[pl.ds(r * VEC, VEC)].astype(jnp.int32)
        sp = jnp.broadcast_to(jnp.sum(vp), (VEC,))
        sc = jnp.broadcast_to(jnp.sum(vi & mask_lo), (VEC,)).astype(jnp.float32)
        st = jnp.broadcast_to(jnp.sum(vi >> TSHIFT), (VEC,)).astype(jnp.float32)
        r16 = jnp.broadcast_to(r, (VEC,))
        plsc.store_scatter(outv, [r16], sp, mask=lane0)
        plsc.store_scatter(outv, [r16 + ROWS], sc, mask=lane0)
        plsc.store_scatter(outv, [r16 + 2 * ROWS], st, mask=lane0)
        return carry

    lax.fori_loop(0, ROWS, _red, 0)
    pltpu.sync_copy(outv, parts_hbm.at[wid])


def _epi_body(parts_ref, w4_ref, k_ref, out_ref):
    e = parts_ref[...]
    a = jnp.dot(w4_ref[...], e, precision=lax.Precision.HIGHEST)
    sp = a[:, :ROWS]
    cnt = a[:, ROWS:2 * ROWS]
    st = a[:, 2 * ROWS:]
    ne = cnt > 0
    safe = jnp.where(ne, cnt, 1.0)
    mp = sp / safe
    mt = st / safe
    diff = jnp.where(ne, jnp.abs(mp - mt), 0.0)
    kk = k_ref[...]
    nv = jnp.maximum(
        jnp.dot(ne.astype(jnp.float32), kk, precision=lax.Precision.HIGHEST),
        1.0)
    dsum = jnp.dot(diff, kk, precision=lax.Precision.HIGHEST)
    stot = jnp.dot(st, kk, precision=lax.Precision.HIGHEST)
    ace = (dsum / nv) * (stot > 0).astype(jnp.float32)
    out_ref[...] = (jnp.sum(ace) / jnp.float32(B * C)).reshape(1, 1)


_epilogue = pl.pallas_call(
    _epi_body,
    out_shape=jax.ShapeDtypeStruct((1, 1), jnp.float32),
)


def kernel(logits, labels):
    lg = logits.reshape(B, C, N)
    lb = labels.reshape(B, N)
    eps = jnp.finfo(jnp.float32).eps
    bounds = jnp.linspace(0.0, 1.0 + eps, N_BINS + 1).astype(jnp.float32)
    step = jnp.float32((1.0 + float(eps)) / 15.0)
    b16 = jnp.float32(16.0) * step
    row_lo = jnp.concatenate([jnp.full((1,), -1.0, jnp.float32), bounds[1:]])
    row_hi = jnp.concatenate([bounds[1:], b16[None]])
    btab = jnp.stack([row_lo, row_hi])
    parts = _sc_hist(lg, lb, btab)
    w4 = (jnp.arange(NW, dtype=jnp.int32)[None, :] // CHUNKS_PER_B
          == jnp.arange(B, dtype=jnp.int32)[:, None]).astype(jnp.float32)
    kk = (jnp.arange(ROWS, dtype=jnp.int32)[:, None] // N_BINS
          == jnp.arange(C, dtype=jnp.int32)[None, :]).astype(jnp.float32)
    return _epilogue(parts, w4, kk).reshape(())
